# initial kernel scaffold (unmeasured)
import jax
import jax.numpy as jnp
from jax import lax
from jax.experimental import pallas as pl
from jax.experimental.pallas import tpu as pltpu

N_Z = 4


def kernel(Q, K, V):
    B, SQ, H, D = Q.shape
    SKV = K.shape[1]
    scale = D ** -0.5

    def body(q_ref, k_ref, v_ref, out_ref,
             comm_o, comm_m, comm_l,
             so_send, so_recv, sm_send, sm_recv, sl_send, sl_recv):
        bi = pl.program_id(0)
        hi = pl.program_id(1)

        q = q_ref[0, :, 0, :]
        k = k_ref[0, :, 0, :]
        v = v_ref[0, :, 0, :]

        s = lax.dot_general(q, k, (((1,), (1,)), ((), ())),
                            preferred_element_type=jnp.float32) * scale
        m = jnp.max(s, axis=1, keepdims=True)
        p = jnp.exp(s - m)
        l = jnp.sum(p, axis=1, keepdims=True)
        o = lax.dot_general(p, v, (((1,), (0,)), ((), ())),
                            preferred_element_type=jnp.float32)

        comm_o[0, bi, :, hi, :] = o
        comm_m[0, bi, :, hi, :] = m
        comm_l[0, bi, :, hi, :] = l

        last = jnp.logical_and(bi == B - 1, hi == H - 1)

        @pl.when(last)
        def _():
            my_x = lax.axis_index("x")
            my_y = lax.axis_index("y")
            my_z = lax.axis_index("z")
            right = (my_x, my_y, lax.rem(my_z + 1, N_Z))
            left = (my_x, my_y, lax.rem(my_z + N_Z - 1, N_Z))

            barrier = pltpu.get_barrier_semaphore()
            for nbr in (left, right):
                pl.semaphore_signal(
                    barrier, inc=1, device_id=nbr,
                    device_id_type=pl.DeviceIdType.MESH,
                )
            pl.semaphore_wait(barrier, 2)

            for hop in range(N_Z - 1):
                rdmas = []
                for buf, ssem, rsem in (
                    (comm_o, so_send, so_recv),
                    (comm_m, sm_send, sm_recv),
                    (comm_l, sl_send, sl_recv),
                ):
                    rd = pltpu.make_async_remote_copy(
                        src_ref=buf.at[hop],
                        dst_ref=buf.at[hop + 1],
                        send_sem=ssem.at[hop],
                        recv_sem=rsem.at[hop],
                        device_id=right,
                        device_id_type=pl.DeviceIdType.MESH,
                    )
                    rd.start()
                    rdmas.append(rd)
                for rd in rdmas:
                    rd.wait()

            m_run = comm_m[0]
            l_run = comm_l[0]
            o_run = comm_o[0]
            for j in range(1, N_Z):
                m_j = comm_m[j]
                m_new = jnp.maximum(m_run, m_j)
                w_a = jnp.exp(m_run - m_new)
                w_b = jnp.exp(m_j - m_new)
                o_run = o_run * w_a + comm_o[j] * w_b
                l_run = l_run * w_a + comm_l[j] * w_b
                m_run = m_new
            out_ref[...] = o_run / l_run

    return pl.pallas_call(
        body,
        grid=(B, H),
        in_specs=[
            pl.BlockSpec((1, SQ, 1, D), lambda b, h: (b, 0, h, 0)),
            pl.BlockSpec((1, SKV, 1, D), lambda b, h: (b, 0, h, 0)),
            pl.BlockSpec((1, SKV, 1, D), lambda b, h: (b, 0, h, 0)),
        ],
        out_specs=pl.BlockSpec((B, SQ, H, D), lambda b, h: (0, 0, 0, 0)),
        out_shape=jax.ShapeDtypeStruct((B, SQ, H, D), jnp.float32),
        scratch_shapes=[
            pltpu.VMEM((N_Z, B, SQ, H, D), jnp.float32),
            pltpu.VMEM((N_Z, B, SQ, H, 1), jnp.float32),
            pltpu.VMEM((N_Z, B, SQ, H, 1), jnp.float32),
            pltpu.SemaphoreType.DMA((N_Z - 1,)),
            pltpu.SemaphoreType.DMA((N_Z - 1,)),
            pltpu.SemaphoreType.DMA((N_Z - 1,)),
            pltpu.SemaphoreType.DMA((N_Z - 1,)),
            pltpu.SemaphoreType.DMA((N_Z - 1,)),
            pltpu.SemaphoreType.DMA((N_Z - 1,)),
        ],
        compiler_params=pltpu.CompilerParams(collective_id=0),
    )(Q, K, V)


# baseline (device time: 148626 ns/iter reference)
import jax
import jax.numpy as jnp
from jax import lax
from jax.experimental import pallas as pl
from jax.experimental.pallas import tpu as pltpu

N_Z = 4


def _drain_pending_transfers():
    import numpy as _np
    try:
        small = _np.zeros(8, _np.float32)
        probes = [jax.device_put(small, d) for d in jax.devices()]
        for p in probes:
            p.block_until_ready()
    except Exception:
        pass


_drain_pending_transfers()


def kernel(Q, K, V):
    B, SQ, H, D = Q.shape
    SKV = K.shape[1]
    scale = D ** -0.5

    def body(q_ref, k_ref, v_ref, out_ref,
             comm_o, comm_m, comm_l,
             so_send, so_recv, sm_send, sm_recv, sl_send, sl_recv):
        bi = pl.program_id(0)

        for h in range(H):
            q = q_ref[0, :, h, :]
            k = k_ref[0, :, h, :]
            v = v_ref[0, :, h, :]

            s = lax.dot_general(q, k, (((1,), (1,)), ((), ())),
                                preferred_element_type=jnp.float32) * scale
            m = jnp.max(s, axis=1, keepdims=True)
            p = jnp.exp(s - m)
            l = jnp.sum(p, axis=1, keepdims=True)
            o = lax.dot_general(p, v, (((1,), (0,)), ((), ())),
                                preferred_element_type=jnp.float32)

            comm_o[0, bi, :, h, :] = o
            comm_m[0, bi, :, h, :] = m
            comm_l[0, bi, :, h, :] = l

        last = bi == B - 1

        @pl.when(last)
        def _():
            my_x = lax.axis_index("x")
            my_y = lax.axis_index("y")
            my_z = lax.axis_index("z")
            right = (my_x, my_y, lax.rem(my_z + 1, N_Z))
            left = (my_x, my_y, lax.rem(my_z + N_Z - 1, N_Z))

            barrier = pltpu.get_barrier_semaphore()
            for nbr in (left, right):
                pl.semaphore_signal(
                    barrier, inc=1, device_id=nbr,
                    device_id_type=pl.DeviceIdType.MESH,
                )
            pl.semaphore_wait(barrier, 2)

            for hop in range(N_Z - 1):
                rdmas = []
                for buf, ssem, rsem in (
                    (comm_o, so_send, so_recv),
                    (comm_m, sm_send, sm_recv),
                    (comm_l, sl_send, sl_recv),
                ):
                    rd = pltpu.make_async_remote_copy(
                        src_ref=buf.at[hop],
                        dst_ref=buf.at[hop + 1],
                        send_sem=ssem.at[hop],
                        recv_sem=rsem.at[hop],
                        device_id=right,
                        device_id_type=pl.DeviceIdType.MESH,
                    )
                    rd.start()
                    rdmas.append(rd)
                for rd in rdmas:
                    rd.wait()

            m_run = comm_m[0]
            l_run = comm_l[0]
            o_run = comm_o[0]
            for j in range(1, N_Z):
                m_j = comm_m[j]
                m_new = jnp.maximum(m_run, m_j)
                w_a = jnp.exp(m_run - m_new)
                w_b = jnp.exp(m_j - m_new)
                o_run = o_run * w_a + comm_o[j] * w_b
                l_run = l_run * w_a + comm_l[j] * w_b
                m_run = m_new
            out_ref[...] = o_run / l_run

    return pl.pallas_call(
        body,
        grid=(B,),
        in_specs=[
            pl.BlockSpec((1, SQ, H, D), lambda b: (b, 0, 0, 0)),
            pl.BlockSpec((1, SKV, H, D), lambda b: (b, 0, 0, 0)),
            pl.BlockSpec((1, SKV, H, D), lambda b: (b, 0, 0, 0)),
        ],
        out_specs=pl.BlockSpec((B, SQ, H, D), lambda b: (0, 0, 0, 0)),
        out_shape=jax.ShapeDtypeStruct((B, SQ, H, D), jnp.float32),
        scratch_shapes=[
            pltpu.VMEM((N_Z, B, SQ, H, D), jnp.float32),
            pltpu.VMEM((N_Z, B, SQ, H, 1), jnp.float32),
            pltpu.VMEM((N_Z, B, SQ, H, 1), jnp.float32),
            pltpu.SemaphoreType.DMA((N_Z - 1,)),
            pltpu.SemaphoreType.DMA((N_Z - 1,)),
            pltpu.SemaphoreType.DMA((N_Z - 1,)),
            pltpu.SemaphoreType.DMA((N_Z - 1,)),
            pltpu.SemaphoreType.DMA((N_Z - 1,)),
            pltpu.SemaphoreType.DMA((N_Z - 1,)),
        ],
        compiler_params=pltpu.CompilerParams(
            collective_id=0, vmem_limit_bytes=100 * 1024 * 1024,
        ),
    )(Q, K, V)
